# Initial kernel scaffold; baseline (speedup 1.0000x reference)
import jax
import jax.numpy as jnp
from jax.experimental import pallas as pl


def kernel(src_nodes, tar_nodes, weights, seed_list):
    raise NotImplementedError("write your pallas kernel here")


# speedup vs baseline: 100.9943x; 100.9943x over previous
import functools

import jax
import jax.numpy as jnp
from jax import lax
from jax.experimental import pallas as pl
from jax.experimental.pallas import tpu as pltpu
from jax.experimental.pallas import tpu_sc as plsc


def _i32(x):
    return jnp.int32(x)


def _fori(n, body):
    lax.fori_loop(jnp.int32(0), jnp.int32(n), body, jnp.int32(0))

N = 100000
NUM_STEPS = 8

NP = 102400
ROWS_N = NP // 128
LANES = 128
CHUNK_ROWS = 16

_mesh = plsc.VectorSubcoreMesh(core_axis_name="c", subcore_axis_name="s")


@functools.partial(
    pl.kernel,
    mesh=_mesh,
    out_type=jax.ShapeDtypeStruct((NP,), jnp.float32),
    scratch_types=[
        pltpu.VMEM((6400,), jnp.float32),
        pltpu.VMEM((1, 128), jnp.float32),
        pltpu.VMEM((1, 128), jnp.int32),
        pltpu.SemaphoreType.DMA,
    ],
)
def _seed_kernel(seed_h, act_h, zero_v, ones_v, idx_v, sem):
    c = lax.axis_index("c")
    s = lax.axis_index("s")

    @pl.when(c == 0)
    def _zero():
        def zfill(k, _):
            zero_v[pl.ds(k * _i32(16), 16)] = jnp.zeros((16,), jnp.float32)
            return jnp.int32(0)

        _fori(6400 // 16, zfill)
        pltpu.sync_copy(zero_v, act_h.at[pl.ds(s * _i32(6400), 6400)])

    plsc.subcore_barrier()

    @pl.when(jnp.logical_and(c == 0, s == 0))
    def _scatter():
        def ofill(k, _):
            ones_v[0, pl.ds(k * _i32(16), 16)] = jnp.ones((16,), jnp.float32)
            return jnp.int32(0)

        _fori(128 // 16, ofill)
        pltpu.sync_copy(seed_h, idx_v)
        pltpu.async_copy(ones_v.at[_i32(0)], act_h.at[idx_v.at[_i32(0)]], sem).wait()


def _phase_a_body(src_h, tar_h, w_h, rand_h, na_h, agg_out_h,
                  agg_sh, zero_v, src_v, tar_v, w_v, rand_v, val_v, succ_v,
                  sem_in, sem_g, sem_s, *, rows_per_w):
    c = lax.axis_index("c")
    s = lax.axis_index("s")
    wid = s * _i32(2) + c

    def zfill(k, _):
        zero_v[pl.ds(k * _i32(16), 16)] = jnp.zeros((16,), jnp.float32)
        return jnp.int32(0)

    _fori(6400 // 16, zfill)
    pltpu.sync_copy(zero_v, agg_sh.at[pl.ds(s * _i32(6400), 6400)])
    plsc.subcore_barrier()

    n_chunks = rows_per_w // CHUNK_ROWS
    row_base = wid * _i32(rows_per_w)

    def chunk(i, _):
        r0 = row_base + i * _i32(CHUNK_ROWS)
        h_in = [
            pltpu.async_copy(src_h.at[pl.ds(r0, CHUNK_ROWS)], src_v, sem_in),
            pltpu.async_copy(tar_h.at[pl.ds(r0, CHUNK_ROWS)], tar_v, sem_in),
            pltpu.async_copy(w_h.at[pl.ds(r0, CHUNK_ROWS)], w_v, sem_in),
            pltpu.async_copy(rand_h.at[pl.ds(r0, CHUNK_ROWS)], rand_v, sem_in),
        ]
        for h in h_in:
            h.wait()
        h_g = [
            pltpu.async_copy(na_h.at[src_v.at[_i32(j)]], val_v.at[_i32(j)], sem_g)
            for j in range(CHUNK_ROWS)
        ]
        for h in h_g:
            h.wait()

        def valu(j, _):
            for k in range(LANES // 16):
                sl = pl.ds(k * 16, 16)
                live = rand_v[j, sl] < w_v[j, sl]
                succ_v[j, sl] = jnp.where(live, val_v[j, sl],
                                          jnp.zeros((16,), jnp.float32))
            return jnp.int32(0)

        _fori(CHUNK_ROWS, valu)
        h_s = [
            pltpu.async_copy(succ_v.at[_i32(j)], agg_sh.at[tar_v.at[_i32(j)]], sem_s,
                             add=True)
            for j in range(CHUNK_ROWS)
        ]
        for h in h_s:
            h.wait()
        return jnp.int32(0)

    _fori(n_chunks, chunk)
    plsc.subcore_barrier()
    pltpu.sync_copy(agg_sh.at[pl.ds(s * _i32(6400), 6400)],
                    agg_out_h.at[c, pl.ds(s * _i32(6400), 6400)])


def _make_phase_a(rows_per_w):
    return functools.partial(
        pl.kernel,
        mesh=_mesh,
        out_type=jax.ShapeDtypeStruct((2, NP), jnp.float32),
        scratch_types=[
            pltpu.VMEM_SHARED((NP,), jnp.float32),
            pltpu.VMEM((6400,), jnp.float32),
            pltpu.VMEM((CHUNK_ROWS, LANES), jnp.int32),
            pltpu.VMEM((CHUNK_ROWS, LANES), jnp.int32),
            pltpu.VMEM((CHUNK_ROWS, LANES), jnp.float32),
            pltpu.VMEM((CHUNK_ROWS, LANES), jnp.float32),
            pltpu.VMEM((CHUNK_ROWS, LANES), jnp.float32),
            pltpu.VMEM((CHUNK_ROWS, LANES), jnp.float32),
            pltpu.SemaphoreType.DMA,
            pltpu.SemaphoreType.DMA,
            pltpu.SemaphoreType.DMA,
        ],
    )(functools.partial(_phase_a_body, rows_per_w=rows_per_w))


def _phase_b_kernel(agg_ref, act_ref, act_out_ref, na_out_ref):
    agg = agg_ref[0] + agg_ref[1]
    sa = jnp.where(agg >= 1.0, jnp.float32(1.0), jnp.float32(0.0))
    act = act_ref[...]
    na = sa * (1.0 - act)
    act_out_ref[...] = act + na
    na_out_ref[...] = na


def _phase_b(aggs, active):
    return pl.pallas_call(
        _phase_b_kernel,
        out_shape=(
            jax.ShapeDtypeStruct((ROWS_N, 128), jnp.float32),
            jax.ShapeDtypeStruct((ROWS_N, 128), jnp.float32),
        ),
    )(aggs, active)


def kernel(src_nodes, tar_nodes, weights, seed_list):
    e = src_nodes.shape[0]
    epw = -(-e // (32 * CHUNK_ROWS * LANES)) * (CHUNK_ROWS * LANES)
    ep = 32 * epw
    rows_per_w = epw // LANES

    src = src_nodes.astype(jnp.int32)
    tar = tar_nodes.astype(jnp.int32)
    w = weights.astype(jnp.float32)
    pad = ep - e
    src = jnp.concatenate([src, jnp.zeros((pad,), jnp.int32)]).reshape(-1, LANES)
    tar = jnp.concatenate([tar, jnp.zeros((pad,), jnp.int32)]).reshape(-1, LANES)
    w = jnp.concatenate([w, jnp.zeros((pad,), jnp.float32)]).reshape(-1, LANES)

    base_key = jax.random.key(12345)
    rands = [
        jnp.concatenate([
            jax.random.uniform(jax.random.fold_in(base_key, step), (e,),
                               dtype=jnp.float32),
            jnp.ones((pad,), jnp.float32),
        ]).reshape(-1, LANES)
        for step in range(NUM_STEPS)
    ]

    seed32 = seed_list.astype(jnp.int32)
    seed_pad = jnp.concatenate(
        [seed32, jnp.full((128 - seed32.shape[0],), seed32[0], jnp.int32)]
    ).reshape(1, 128)

    active_flat = _seed_kernel(seed_pad)
    phase_a = _make_phase_a(rows_per_w)

    active = active_flat.reshape(ROWS_N, 128)
    new_active_flat = active_flat
    for step in range(NUM_STEPS):
        aggs = phase_a(src, tar, w, rands[step], new_active_flat)
        active, new_active = _phase_b(aggs.reshape(2, ROWS_N, 128), active)
        new_active_flat = new_active.reshape(NP)

    return active.reshape(NP)[:N]


# speedup vs baseline: 287.5613x; 2.8473x over previous
import functools

import jax
import jax.numpy as jnp
from jax import lax
from jax.experimental import pallas as pl
from jax.experimental.pallas import tpu as pltpu
from jax.experimental.pallas import tpu_sc as plsc


def _i32(x):
    return jnp.int32(x)


def _fori(n, body):
    lax.fori_loop(jnp.int32(0), jnp.int32(n), body, jnp.int32(0))


N = 100000
NUM_STEPS = 8

NP = 102400
U = NP // (32 * 128)
NW = U * 128
LANES = 128
CHUNK_ROWS = 16

_mesh = plsc.VectorSubcoreMesh(core_axis_name="c", subcore_axis_name="s")


@functools.partial(
    pl.kernel,
    mesh=_mesh,
    out_type=jax.ShapeDtypeStruct((NP,), jnp.float32),
    scratch_types=[
        pltpu.VMEM((6400,), jnp.float32),
        pltpu.VMEM((1, 128), jnp.float32),
        pltpu.VMEM((1, 128), jnp.int32),
        pltpu.SemaphoreType.DMA,
    ],
)
def _seed_kernel(seed_h, act_h, zero_v, ones_v, idx_v, sem):
    c = lax.axis_index("c")
    s = lax.axis_index("s")

    @pl.when(c == 0)
    def _zero():
        def zfill(k, _):
            zero_v[pl.ds(k * _i32(16), 16)] = jnp.zeros((16,), jnp.float32)
            return jnp.int32(0)

        _fori(6400 // 16, zfill)
        pltpu.sync_copy(zero_v, act_h.at[pl.ds(s * _i32(6400), 6400)])

    plsc.subcore_barrier()

    @pl.when(jnp.logical_and(c == 0, s == 0))
    def _scatter():
        def ofill(k, _):
            ones_v[0, pl.ds(k * _i32(16), 16)] = jnp.ones((16,), jnp.float32)
            return jnp.int32(0)

        _fori(128 // 16, ofill)
        pltpu.sync_copy(seed_h, idx_v)
        pltpu.async_copy(ones_v.at[_i32(0)], act_h.at[idx_v.at[_i32(0)]], sem).wait()


def _phase_a_body(src_h, tar_h, w_h, rand_h, bits_h, agg_out_h,
                  agg_sh, zero_v, bits_v,
                  src_a, tar_a, w_a, rand_a, succ_a,
                  src_b, tar_b, w_b, rand_b, succ_b,
                  sem_a, sem_b, sem_s, *, rows_per_w):
    c = lax.axis_index("c")
    s = lax.axis_index("s")
    wid = s * _i32(2) + c
    n_chunks = rows_per_w // CHUNK_ROWS
    row_base = wid * _i32(rows_per_w)

    bufs = ((src_a, tar_a, w_a, rand_a, succ_a, sem_a),
            (src_b, tar_b, w_b, rand_b, succ_b, sem_b))

    pltpu.sync_copy(bits_h, bits_v)

    def zfill(k, _):
        zero_v[pl.ds(k * _i32(16), 16)] = jnp.zeros((16,), jnp.float32)
        return jnp.int32(0)

    _fori(6400 // 16, zfill)
    pltpu.sync_copy(zero_v, agg_sh.at[pl.ds(s * _i32(6400), 6400)])
    plsc.subcore_barrier()

    def issue(i, buf):
        sv, tv, wv, rv, _, sem = buf
        r0 = row_base + i * _i32(CHUNK_ROWS)
        pltpu.async_copy(src_h.at[pl.ds(r0, CHUNK_ROWS)], sv, sem)
        pltpu.async_copy(tar_h.at[pl.ds(r0, CHUNK_ROWS)], tv, sem)
        pltpu.async_copy(w_h.at[pl.ds(r0, CHUNK_ROWS)], wv, sem)
        pltpu.async_copy(rand_h.at[pl.ds(r0, CHUNK_ROWS)], rv, sem)

    def drain_inputs(buf):
        sv, tv, wv, rv, _, sem = buf
        pltpu.make_async_copy(src_h.at[pl.ds(_i32(0), CHUNK_ROWS)], sv, sem).wait()
        pltpu.make_async_copy(tar_h.at[pl.ds(_i32(0), CHUNK_ROWS)], tv, sem).wait()
        pltpu.make_async_copy(w_h.at[pl.ds(_i32(0), CHUNK_ROWS)], wv, sem).wait()
        pltpu.make_async_copy(rand_h.at[pl.ds(_i32(0), CHUNK_ROWS)], rv, sem).wait()

    def process(i, buf, other):
        sv, tv, wv, rv, succv, _ = buf
        drain_inputs(buf)

        @pl.when(i + _i32(1) < _i32(n_chunks))
        def _issue_next():
            issue(i + _i32(1), other)

        def valu(j, acc):
            for k8 in range(LANES // 16):
                sl = pl.ds(k8 * 16, 16)
                n = sv[j, sl]
                widx = ((n >> 12) << 7) | (n & 127)
                word = plsc.load_gather(bits_v, [widx])
                bit = (word >> ((n >> 7) & 31)) & 1
                live = rv[j, sl] < wv[j, sl]
                fire = jnp.where(live, bit, jnp.zeros((16,), jnp.int32))
                firef = fire.astype(jnp.float32)
                succv[j, sl] = firef
                acc = acc + firef
            return acc

        acc = lax.fori_loop(jnp.int32(0), jnp.int32(CHUNK_ROWS), valu,
                            jnp.zeros((16,), jnp.float32))
        total = jnp.sum(acc)

        @pl.when(total > 0.0)
        def _scatter():
            hs = [
                pltpu.async_copy(succv.at[_i32(j)], agg_sh.at[tv.at[_i32(j)]],
                                 sem_s, add=True)
                for j in range(CHUNK_ROWS)
            ]
            for h in hs:
                h.wait()

    issue(_i32(0), bufs[0])

    def pair(ii, _):
        process(ii * _i32(2), bufs[0], bufs[1])
        process(ii * _i32(2) + _i32(1), bufs[1], bufs[0])
        return jnp.int32(0)

    _fori(n_chunks // 2, pair)
    plsc.subcore_barrier()
    pltpu.sync_copy(agg_sh.at[pl.ds(s * _i32(6400), 6400)],
                    agg_out_h.at[c, pl.ds(s * _i32(6400), 6400)])


def _make_phase_a(rows_per_w):
    edge_buf = lambda dt: pltpu.VMEM((CHUNK_ROWS, LANES), dt)
    return functools.partial(
        pl.kernel,
        mesh=_mesh,
        out_type=jax.ShapeDtypeStruct((2, NP), jnp.float32),
        compiler_params=pltpu.CompilerParams(needs_layout_passes=False),
        scratch_types=[
            pltpu.VMEM_SHARED((NP,), jnp.float32),
            pltpu.VMEM((6400,), jnp.float32),
            pltpu.VMEM((NW,), jnp.int32),
            edge_buf(jnp.int32), edge_buf(jnp.int32),
            edge_buf(jnp.float32), edge_buf(jnp.float32), edge_buf(jnp.float32),
            edge_buf(jnp.int32), edge_buf(jnp.int32),
            edge_buf(jnp.float32), edge_buf(jnp.float32), edge_buf(jnp.float32),
            pltpu.SemaphoreType.DMA,
            pltpu.SemaphoreType.DMA,
            pltpu.SemaphoreType.DMA,
        ],
    )(functools.partial(_phase_a_body, rows_per_w=rows_per_w))


def _pack_bits(na):
    k = lax.broadcasted_iota(jnp.int32, (U, 32, 128), 1)
    return jnp.sum(na.astype(jnp.int32) << k, axis=1, dtype=jnp.int32)


def _phase_b_kernel(agg_ref, act_ref, act_out_ref, bits_out_ref):
    agg = agg_ref[0] + agg_ref[1]
    sa = jnp.where(agg >= 1.0, jnp.float32(1.0), jnp.float32(0.0))
    act = act_ref[...]
    na = sa * (1.0 - act)
    act_out_ref[...] = act + na
    bits_out_ref[...] = _pack_bits(na)


def _phase_b(aggs, active):
    return pl.pallas_call(
        _phase_b_kernel,
        out_shape=(
            jax.ShapeDtypeStruct((U, 32, 128), jnp.float32),
            jax.ShapeDtypeStruct((U, 128), jnp.int32),
        ),
    )(aggs, active)


def _pack_kernel_body(act_ref, bits_out_ref):
    bits_out_ref[...] = _pack_bits(act_ref[...])


def _pack_seed_bits(active):
    return pl.pallas_call(
        _pack_kernel_body,
        out_shape=jax.ShapeDtypeStruct((U, 128), jnp.int32),
    )(active)


def kernel(src_nodes, tar_nodes, weights, seed_list):
    e = src_nodes.shape[0]
    cpw = -(-e // (32 * 2 * CHUNK_ROWS * LANES)) * 2
    epw = cpw * CHUNK_ROWS * LANES
    ep = 32 * epw
    rows_per_w = epw // LANES

    src = src_nodes.astype(jnp.int32)
    tar = tar_nodes.astype(jnp.int32)
    w = weights.astype(jnp.float32)
    pad = ep - e
    src = jnp.concatenate([src, jnp.zeros((pad,), jnp.int32)]).reshape(-1, LANES)
    tar = jnp.concatenate([tar, jnp.zeros((pad,), jnp.int32)]).reshape(-1, LANES)
    w = jnp.concatenate([w, jnp.zeros((pad,), jnp.float32)]).reshape(-1, LANES)

    base_key = jax.random.key(12345)
    rands = [
        jnp.concatenate([
            jax.random.uniform(jax.random.fold_in(base_key, step), (e,),
                               dtype=jnp.float32),
            jnp.ones((pad,), jnp.float32),
        ]).reshape(-1, LANES)
        for step in range(NUM_STEPS)
    ]

    seed32 = seed_list.astype(jnp.int32)
    seed_pad = jnp.concatenate(
        [seed32, jnp.full((128 - seed32.shape[0],), seed32[0], jnp.int32)]
    ).reshape(1, 128)

    active_flat = _seed_kernel(seed_pad)
    phase_a = _make_phase_a(rows_per_w)

    active = active_flat.reshape(U, 32, 128)
    bits = _pack_seed_bits(active)
    for step in range(NUM_STEPS):
        aggs = phase_a(src, tar, w, rands[step], bits.reshape(NW))
        active, bits = _phase_b(aggs.reshape(2, U, 32, 128), active)

    return active.reshape(NP)[:N]
